# baseline (speedup 1.0000x reference)
import math

import jax
import jax.numpy as jnp
from jax import lax
from jax.experimental import pallas as pl
from jax.experimental.pallas import tpu as pltpu
from jax.experimental.pallas import tpu_sc as plsc

NIMG = 16
HE = 512
WE = 1024
B = 16
CROP = 512
HW = CROP * CROP
NPIX = B * HW
NROW = B * CROP
HBLK = 64
NHB = CROP // HBLK
TAB_ROWS = NIMG * HE * WE
ROWD = 8

NC = 2
NS = 16
NW = NC * NS
ROWS_PER_TILE = NROW // NW


def _table_body(img_ref, ima_ref, dis_ref, dma_ref, out_ref):
    im3 = img_ref[0]
    arr8 = jnp.concatenate([
        im3, ima_ref[0], dis_ref[0], dma_ref[0],
        jnp.zeros((2, HBLK, WE), jnp.float32)], axis=0)
    t = jnp.transpose(arr8, (1, 2, 0))
    out_ref[0] = t.reshape(HBLK, WE * ROWD)


def _table_call(image, image_mask, distance, distance_mask, interpret=False):
    out = pl.pallas_call(
        _table_body,
        grid=(NIMG, HE // HBLK),
        in_specs=[
            pl.BlockSpec((1, 3, HBLK, WE), lambda n, h: (n, 0, h, 0)),
            pl.BlockSpec((1, 1, HBLK, WE), lambda n, h: (n, 0, h, 0)),
            pl.BlockSpec((1, 1, HBLK, WE), lambda n, h: (n, 0, h, 0)),
            pl.BlockSpec((1, 1, HBLK, WE), lambda n, h: (n, 0, h, 0)),
        ],
        out_specs=pl.BlockSpec((1, HBLK, WE * ROWD), lambda n, h: (n * (HE // HBLK) + h, 0, 0)),
        out_shape=jax.ShapeDtypeStruct((NIMG * (HE // HBLK), HBLK, WE * ROWD), jnp.float32),
        compiler_params=pltpu.CompilerParams(
            dimension_semantics=("parallel", "parallel")),
        interpret=interpret,
    )(image, image_mask, distance, distance_mask)
    return out.reshape(TAB_ROWS, ROWD)


def _bf(x):
    return x.astype(jnp.bfloat16).astype(jnp.float32)


def _bf_hard(x):
    bits = lax.bitcast_convert_type(x, jnp.uint32)
    bits = (bits + jnp.uint32(0x7FFF) + ((bits >> 16) & jnp.uint32(1))) & jnp.uint32(0xFFFF0000)
    return lax.bitcast_convert_type(bits, jnp.float32)


def _kahan3(p0, p1, p2):
    s1 = p0 + p1
    bp = s1 - p0
    e1 = (p0 - (s1 - bp)) + (p1 - bp)
    s2 = s1 + p2
    bp2 = s2 - s1
    e2 = (s1 - (s2 - bp2)) + (p2 - bp2)
    return s2 + (e1 + e2)


def _taps_body(scal_ref, rot_ref, base_ref, uv_ref, idx_ref, w_ref):
    b = pl.program_id(0)
    hb = pl.program_id(1)
    tx = scal_ref[0]
    ty = scal_ref[1]
    wf = scal_ref[2]
    hf = scal_ref[3]
    ii = lax.broadcasted_iota(jnp.int32, (HBLK, CROP), 0).astype(jnp.float32) + (hb * HBLK).astype(jnp.float32)
    jj = lax.broadcasted_iota(jnp.int32, (HBLK, CROP), 1).astype(jnp.float32)
    ux = (jj + 0.5) / wf * 2.0 - 1.0
    uy = (ii + 0.5) / hf * 2.0 - 1.0
    x = tx * ux
    y = ty * uy
    z = jnp.ones((HBLK, CROP), jnp.float32)
    n = jnp.sqrt((x * x + y * y) + z * z)
    dnx = _bf(x / n)
    dny = _bf(y / n)
    dnz = _bf(z / n)
    r00 = rot_ref[b, 0]; r01 = rot_ref[b, 1]; r02 = rot_ref[b, 2]
    r10 = rot_ref[b, 3]; r11 = rot_ref[b, 4]; r12 = rot_ref[b, 5]
    r20 = rot_ref[b, 6]; r21 = rot_ref[b, 7]; r22 = rot_ref[b, 8]
    dx = _kahan3(r00 * dnx, r01 * dny, r02 * dnz)
    dy = _kahan3(r10 * dnx, r11 * dny, r12 * dnz)
    dz = _kahan3(r20 * dnx, r21 * dny, r22 * dnz)
    theta = jnp.arctan2(dx, dz)
    sphi = jnp.clip(dy, -1.0, 1.0)
    phi = 2.0 * jnp.arctan2(sphi, 1.0 + jnp.sqrt((1.0 - sphi) * (1.0 + sphi)))
    u = -theta / math.pi
    v = 2.0 * phi / math.pi
    uv_ref[0] = jnp.stack([u, v], axis=-1).reshape(HBLK, 2 * CROP)

    ix = ((u + 1.0) * WE - 1.0) * 0.5
    iy = ((v + 1.0) * HE - 1.0) * 0.5
    ix0f = jnp.floor(ix)
    iy0f = jnp.floor(iy)
    wx = ix - ix0f
    wy = iy - iy0f
    ix0 = ix0f.astype(jnp.int32)
    iy0 = iy0f.astype(jnp.int32)
    ix1 = ix0 + 1
    iy1 = iy0 + 1
    vx0 = ((ix0 >= 0) & (ix0 < WE)).astype(jnp.float32)
    vx1 = ((ix1 >= 0) & (ix1 < WE)).astype(jnp.float32)
    vy0 = ((iy0 >= 0) & (iy0 < HE)).astype(jnp.float32)
    vy1 = ((iy1 >= 0) & (iy1 < HE)).astype(jnp.float32)
    ax0 = (1.0 - wx) * vx0
    ax1 = wx * vx1
    ay0 = (1.0 - wy) * vy0
    ay1 = wy * vy1
    ix0c = jnp.clip(ix0, 0, WE - 1)
    ix1c = jnp.clip(ix1, 0, WE - 1)
    iy0c = jnp.clip(iy0, 0, HE - 1)
    iy1c = jnp.clip(iy1, 0, HE - 1)
    base = base_ref[b]
    i00 = base + iy0c * WE + ix0c
    i01 = base + iy0c * WE + ix1c
    i10 = base + iy1c * WE + ix0c
    i11 = base + iy1c * WE + ix1c
    idx_ref[...] = jnp.stack([i00, i01, i10, i11], axis=0)
    w_ref[...] = jnp.stack([ax0 * ay0, ax1 * ay0, ax0 * ay1, ax1 * ay1], axis=0)


def _taps_call(scal, rot, base, interpret=False):
    return pl.pallas_call(
        _taps_body,
        grid=(B, NHB),
        in_specs=[
            pl.BlockSpec(memory_space=pltpu.SMEM),
            pl.BlockSpec(memory_space=pltpu.SMEM),
            pl.BlockSpec(memory_space=pltpu.SMEM),
        ],
        out_specs=[
            pl.BlockSpec((1, HBLK, 2 * CROP), lambda b, h: (b, h, 0)),
            pl.BlockSpec((4, HBLK, CROP), lambda b, h: (0, b * NHB + h, 0)),
            pl.BlockSpec((4, HBLK, CROP), lambda b, h: (0, b * NHB + h, 0)),
        ],
        out_shape=[
            jax.ShapeDtypeStruct((B, CROP, 2 * CROP), jnp.float32),
            jax.ShapeDtypeStruct((4, NROW, CROP), jnp.int32),
            jax.ShapeDtypeStruct((4, NROW, CROP), jnp.float32),
        ],
        compiler_params=pltpu.CompilerParams(
            dimension_semantics=("parallel", "parallel")),
        interpret=interpret,
    )(scal, rot, base)


def _inv_body(scal_ref, rot_ref, uv_ref):
    b = pl.program_id(0)
    hb = pl.program_id(1)
    inv = scal_ref[4]
    ii = lax.broadcasted_iota(jnp.int32, (HBLK, WE), 0).astype(jnp.float32) + (hb * HBLK).astype(jnp.float32)
    jj = lax.broadcasted_iota(jnp.int32, (HBLK, WE), 1).astype(jnp.float32)
    ue = (jj + 0.5) / WE * 2.0 - 1.0
    ve = (ii + 0.5) / HE * 2.0 - 1.0
    th = -ue * math.pi
    ph = (ve * math.pi) / 2.0
    sph = jnp.sin(ph)
    cph = jnp.cos(ph)
    sth = jnp.sin(th)
    cth = jnp.cos(th)
    ex = _bf(sth * cph)
    ey = _bf(sph)
    ez = _bf(cth * cph)
    r00 = rot_ref[b, 0]; r01 = rot_ref[b, 1]; r02 = rot_ref[b, 2]
    r10 = rot_ref[b, 3]; r11 = rot_ref[b, 4]; r12 = rot_ref[b, 5]
    r20 = rot_ref[b, 6]; r21 = rot_ref[b, 7]; r22 = rot_ref[b, 8]
    x = _kahan3(r00 * ex, r10 * ey, r20 * ez)
    y = _kahan3(r01 * ex, r11 * ey, r21 * ez)
    z = _kahan3(r02 * ex, r12 * ey, r22 * ez)
    eps = 1e-6
    valid = z > eps
    zc = jnp.where(valid, z, eps)
    ug = jnp.where(valid, x / zc, inv)
    vg = jnp.where(valid, y / zc, inv)
    uv_ref[0] = jnp.stack([ug, vg], axis=-1).reshape(HBLK, 2 * WE)


def _inv_call(scal, rot, interpret=False):
    return pl.pallas_call(
        _inv_body,
        grid=(B, HE // HBLK),
        in_specs=[
            pl.BlockSpec(memory_space=pltpu.SMEM),
            pl.BlockSpec(memory_space=pltpu.SMEM),
        ],
        out_specs=[
            pl.BlockSpec((1, HBLK, 2 * WE), lambda b, h: (b, h, 0)),
        ],
        out_shape=[
            jax.ShapeDtypeStruct((B, HE, 2 * WE), jnp.float32),
        ],
        compiler_params=pltpu.CompilerParams(
            dimension_semantics=("parallel", "parallel")),
        interpret=interpret,
    )(scal, rot)


def _sc_body(table, idxh, wh, img_o, ima_o, dis_o, dma_o,
             iw, wv, gbuf, obuf, gsem, psem, osem):
    cid = lax.axis_index("c")
    sid = lax.axis_index("s")
    wid = sid * NC + cid
    b = wid // 2
    row0 = wid * ROWS_PER_TILE
    last = ROWS_PER_TILE - 1
    lane = lax.broadcasted_iota(jnp.int32, (16,), 0)

    def prefetch(k, slot):
        rowg = row0 + k
        a = pltpu.async_copy(idxh.at[:, rowg, :], iw.at[slot], psem)
        bcp = pltpu.async_copy(wh.at[:, rowg, :], wv.at[slot], psem)
        return a, bcp

    def issue_gathers(slot):
        cps = []
        for t in range(4):
            for j in range(4):
                cps.append(pltpu.async_copy(
                    table.at[iw.at[slot, t, pl.ds(j * 128, 128)]],
                    gbuf.at[slot, t].at[pl.ds(j * 128, 128)],
                    gsem))
        return cps

    pltpu.sync_copy(idxh.at[:, row0, :], iw.at[0])
    pltpu.sync_copy(wh.at[:, row0, :], wv.at[0])
    issue_gathers(0)
    prefetch(jnp.minimum(1, last), 1)

    def process(k, cur, nxt):
        q = (row0 + k - b * CROP) * CROP
        for _ in range(16):
            pltpu.make_async_copy(table.at[pl.ds(0, 128)],
                                  gbuf.at[0, 0].at[pl.ds(0, 128)], gsem).wait()
        pltpu.make_async_copy(idxh.at[:, 0, :], iw.at[0], psem).wait()
        pltpu.make_async_copy(wh.at[:, 0, :], wv.at[0], psem).wait()
        issue_gathers(nxt)
        for p16 in range(CROP // 16):
            w0 = wv[cur, 0, pl.ds(p16 * 16, 16)]
            w1 = wv[cur, 1, pl.ds(p16 * 16, 16)]
            w2 = wv[cur, 2, pl.ds(p16 * 16, 16)]
            w3 = wv[cur, 3, pl.ds(p16 * 16, 16)]
            pidx = lane + (p16 * 16)
            gcur = gbuf.at[cur]
            for c in range(6):
                cc = jnp.full((16,), c, jnp.int32)
                g0 = plsc.load_gather(gcur, [jnp.full((16,), 0, jnp.int32), pidx, cc])
                g1 = plsc.load_gather(gcur, [jnp.full((16,), 1, jnp.int32), pidx, cc])
                g2 = plsc.load_gather(gcur, [jnp.full((16,), 2, jnp.int32), pidx, cc])
                g3 = plsc.load_gather(gcur, [jnp.full((16,), 3, jnp.int32), pidx, cc])
                acc = w0 * g0 + w1 * g1 + w2 * g2 + w3 * g3
                obuf[cur, c, pl.ds(p16 * 16, 16)] = acc
        prefetch(jnp.minimum(k + 2, last), cur)
        @pl.when(k > 0)
        def _():
            for c in range(6):
                pltpu.make_async_copy(wh.at[0, 0, :], obuf.at[0, 0], osem).wait()
        pltpu.async_copy(obuf.at[cur, 0], img_o.at[pl.ds((b * 3 + 0) * HW + q, CROP)], osem)
        pltpu.async_copy(obuf.at[cur, 1], img_o.at[pl.ds((b * 3 + 1) * HW + q, CROP)], osem)
        pltpu.async_copy(obuf.at[cur, 2], img_o.at[pl.ds((b * 3 + 2) * HW + q, CROP)], osem)
        pltpu.async_copy(obuf.at[cur, 3], ima_o.at[pl.ds(b * HW + q, CROP)], osem)
        pltpu.async_copy(obuf.at[cur, 4], dis_o.at[pl.ds(b * HW + q, CROP)], osem)
        pltpu.async_copy(obuf.at[cur, 5], dma_o.at[pl.ds(b * HW + q, CROP)], osem)

    def body(m, carry):
        process(2 * m, 0, 1)
        process(2 * m + 1, 1, 0)
        return carry

    lax.fori_loop(0, ROWS_PER_TILE // 2, body, 0)
    pltpu.make_async_copy(idxh.at[:, 0, :], iw.at[0], psem).wait()
    pltpu.make_async_copy(wh.at[:, 0, :], wv.at[0], psem).wait()
    for _ in range(16):
        pltpu.make_async_copy(table.at[pl.ds(0, 128)],
                              gbuf.at[0, 0].at[pl.ds(0, 128)], gsem).wait()
    for _ in range(6):
        pltpu.make_async_copy(wh.at[0, 0, :], obuf.at[0, 0], osem).wait()


def _sc_call(table, idxh, wh, interpret=False):
    mesh = plsc.VectorSubcoreMesh(core_axis_name="c", subcore_axis_name="s",
                                  num_cores=NC, num_subcores=NS)
    fn = pl.kernel(
        _sc_body,
        out_type=[
            jax.ShapeDtypeStruct((B * 3 * HW,), jnp.float32),
            jax.ShapeDtypeStruct((NPIX,), jnp.float32),
            jax.ShapeDtypeStruct((NPIX,), jnp.float32),
            jax.ShapeDtypeStruct((NPIX,), jnp.float32),
        ],
        mesh=mesh,
        scratch_types=[
            pltpu.VMEM((2, 4, CROP), jnp.int32),
            pltpu.VMEM((2, 4, CROP), jnp.float32),
            pltpu.VMEM((2, 4, CROP, ROWD), jnp.float32),
            pltpu.VMEM((2, 6, CROP), jnp.float32),
            pltpu.SemaphoreType.DMA,
            pltpu.SemaphoreType.DMA,
            pltpu.SemaphoreType.DMA,
        ],
        compiler_params=pltpu.CompilerParams(needs_layout_passes=False,
                                             use_tc_tiling_on_sc=False),
        interpret=interpret,
    )
    return fn(table, idxh, wh)


def kernel(image_indices, yaws, pitches, fov_x, fov_y, width, height,
           image, image_mask, distance, distance_mask, invalid_number=2.0):
    f32 = jnp.float32
    yr = (yaws.astype(f32) * math.pi) / 180.0
    pr = ((-1.0 * pitches.astype(f32)) * math.pi) / 180.0
    cy, sy = jnp.cos(yr), jnp.sin(yr)
    cp, sp = jnp.cos(pr), jnp.sin(pr)
    bcy, bsy, bcp, bsp = _bf_hard(cy), _bf_hard(sy), _bf_hard(cp), _bf_hard(sp)
    zero = jnp.zeros_like(cp)
    rot = jnp.stack([bcy * bcp, -bsy, bcy * bsp,
                     bsy * bcp, bcy, bsy * bsp,
                     -bsp, zero, bcp], axis=-1)
    rot_bf = _bf_hard(rot)
    fovx_r = (jnp.asarray(fov_x, f32) * math.pi) / 180.0
    fovy_r = (jnp.asarray(fov_y, f32) * math.pi) / 180.0
    scal = jnp.stack([
        jnp.tan(fovx_r / 2.0),
        jnp.tan(fovy_r / 2.0),
        jnp.asarray(width, f32),
        jnp.asarray(height, f32),
        jnp.asarray(invalid_number, f32),
        jnp.zeros((), f32), jnp.zeros((), f32), jnp.zeros((), f32),
    ])
    base = image_indices.astype(jnp.int32) * (HE * WE)

    table = _table_call(image.astype(f32), image_mask.astype(f32),
                        distance.astype(f32), distance_mask.astype(f32))

    uv, idxh, wh = _taps_call(scal, rot_bf, base)
    img_f, ima_f, dis_f, dma_f = _sc_call(table, idxh, wh)
    (iuv,) = _inv_call(scal, rot_bf)

    image_out = img_f.reshape(B, 3, CROP, CROP)
    image_mask_out = ima_f.reshape(B, 1, CROP, CROP)
    distance_out = dis_f.reshape(B, 1, CROP, CROP)
    distance_mask_out = dma_f.reshape(B, 1, CROP, CROP)
    grid = uv.reshape(B, CROP, CROP, 2)
    inverse_grid = iuv.reshape(B, HE, WE, 2)
    return (image_out, image_mask_out, distance_out, distance_mask_out,
            grid, inverse_grid)


# speedup vs baseline: 1.9117x; 1.9117x over previous
import math

import jax
import jax.numpy as jnp
from jax import lax
from jax.experimental import pallas as pl
from jax.experimental.pallas import tpu as pltpu
from jax.experimental.pallas import tpu_sc as plsc

NIMG = 16
HE = 512
WE = 1024
B = 16
CROP = 512
HW = CROP * CROP
NPIX = B * HW
NROW = B * CROP
HBLK = 64
NHB = CROP // HBLK
TAB_ROWS = NIMG * HE * WE
ROWD = 8

NC = 2
NS = 16
NW = NC * NS
ROWS_PER_TILE = NROW // NW


def _table_body(img_ref, ima_ref, dis_ref, dma_ref, out_ref):
    im3 = img_ref[0]
    arr8 = jnp.concatenate([
        im3, ima_ref[0], dis_ref[0], dma_ref[0],
        jnp.zeros((2, HBLK, WE), jnp.float32)], axis=0)
    t = jnp.transpose(arr8, (1, 2, 0))
    out_ref[0] = t.reshape(HBLK, WE * ROWD)


def _table_call(image, image_mask, distance, distance_mask, interpret=False):
    out = pl.pallas_call(
        _table_body,
        grid=(NIMG, HE // HBLK),
        in_specs=[
            pl.BlockSpec((1, 3, HBLK, WE), lambda n, h: (n, 0, h, 0)),
            pl.BlockSpec((1, 1, HBLK, WE), lambda n, h: (n, 0, h, 0)),
            pl.BlockSpec((1, 1, HBLK, WE), lambda n, h: (n, 0, h, 0)),
            pl.BlockSpec((1, 1, HBLK, WE), lambda n, h: (n, 0, h, 0)),
        ],
        out_specs=pl.BlockSpec((1, HBLK, WE * ROWD), lambda n, h: (n * (HE // HBLK) + h, 0, 0)),
        out_shape=jax.ShapeDtypeStruct((NIMG * (HE // HBLK), HBLK, WE * ROWD), jnp.float32),
        compiler_params=pltpu.CompilerParams(
            dimension_semantics=("parallel", "parallel")),
        interpret=interpret,
    )(image, image_mask, distance, distance_mask)
    return out.reshape(TAB_ROWS, ROWD)


def _bf(x):
    return x.astype(jnp.bfloat16).astype(jnp.float32)


def _bf_hard(x):
    bits = lax.bitcast_convert_type(x, jnp.uint32)
    bits = (bits + jnp.uint32(0x7FFF) + ((bits >> 16) & jnp.uint32(1))) & jnp.uint32(0xFFFF0000)
    return lax.bitcast_convert_type(bits, jnp.float32)


def _kahan3(p0, p1, p2):
    s1 = p0 + p1
    bp = s1 - p0
    e1 = (p0 - (s1 - bp)) + (p1 - bp)
    s2 = s1 + p2
    bp2 = s2 - s1
    e2 = (s1 - (s2 - bp2)) + (p2 - bp2)
    return s2 + (e1 + e2)


def _taps_body(scal_ref, rot_ref, base_ref, u_ref, v_ref, idx_ref, w_ref):
    b = pl.program_id(0)
    hb = pl.program_id(1)
    tx = scal_ref[0]
    ty = scal_ref[1]
    wf = scal_ref[2]
    hf = scal_ref[3]
    ii = lax.broadcasted_iota(jnp.int32, (HBLK, CROP), 0).astype(jnp.float32) + (hb * HBLK).astype(jnp.float32)
    jj = lax.broadcasted_iota(jnp.int32, (HBLK, CROP), 1).astype(jnp.float32)
    ux = (jj + 0.5) / wf * 2.0 - 1.0
    uy = (ii + 0.5) / hf * 2.0 - 1.0
    x = tx * ux
    y = ty * uy
    z = jnp.ones((HBLK, CROP), jnp.float32)
    n = jnp.sqrt((x * x + y * y) + z * z)
    dnx = _bf(x / n)
    dny = _bf(y / n)
    dnz = _bf(z / n)
    r00 = rot_ref[b, 0]; r01 = rot_ref[b, 1]; r02 = rot_ref[b, 2]
    r10 = rot_ref[b, 3]; r11 = rot_ref[b, 4]; r12 = rot_ref[b, 5]
    r20 = rot_ref[b, 6]; r21 = rot_ref[b, 7]; r22 = rot_ref[b, 8]
    dx = _kahan3(r00 * dnx, r01 * dny, r02 * dnz)
    dy = _kahan3(r10 * dnx, r11 * dny, r12 * dnz)
    dz = _kahan3(r20 * dnx, r21 * dny, r22 * dnz)
    theta = jnp.arctan2(dx, dz)
    sphi = jnp.clip(dy, -1.0, 1.0)
    phi = 2.0 * jnp.arctan2(sphi, 1.0 + jnp.sqrt((1.0 - sphi) * (1.0 + sphi)))
    u = -theta / math.pi
    v = 2.0 * phi / math.pi
    u_ref[0] = u
    v_ref[0] = v

    ix = ((u + 1.0) * WE - 1.0) * 0.5
    iy = ((v + 1.0) * HE - 1.0) * 0.5
    ix0f = jnp.floor(ix)
    iy0f = jnp.floor(iy)
    wx = ix - ix0f
    wy = iy - iy0f
    ix0 = ix0f.astype(jnp.int32)
    iy0 = iy0f.astype(jnp.int32)
    ix1 = ix0 + 1
    iy1 = iy0 + 1
    vx0 = ((ix0 >= 0) & (ix0 < WE)).astype(jnp.float32)
    vx1 = ((ix1 >= 0) & (ix1 < WE)).astype(jnp.float32)
    vy0 = ((iy0 >= 0) & (iy0 < HE)).astype(jnp.float32)
    vy1 = ((iy1 >= 0) & (iy1 < HE)).astype(jnp.float32)
    ax0 = (1.0 - wx) * vx0
    ax1 = wx * vx1
    ay0 = (1.0 - wy) * vy0
    ay1 = wy * vy1
    ix0c = jnp.clip(ix0, 0, WE - 1)
    ix1c = jnp.clip(ix1, 0, WE - 1)
    iy0c = jnp.clip(iy0, 0, HE - 1)
    iy1c = jnp.clip(iy1, 0, HE - 1)
    base = base_ref[b]
    i00 = base + iy0c * WE + ix0c
    i01 = base + iy0c * WE + ix1c
    i10 = base + iy1c * WE + ix0c
    i11 = base + iy1c * WE + ix1c
    idx_ref[...] = jnp.stack([i00, i01, i10, i11], axis=0)
    w_ref[...] = jnp.stack([ax0 * ay0, ax1 * ay0, ax0 * ay1, ax1 * ay1], axis=0)


def _taps_call(scal, rot, base, interpret=False):
    return pl.pallas_call(
        _taps_body,
        grid=(B, NHB),
        in_specs=[
            pl.BlockSpec(memory_space=pltpu.SMEM),
            pl.BlockSpec(memory_space=pltpu.SMEM),
            pl.BlockSpec(memory_space=pltpu.SMEM),
        ],
        out_specs=[
            pl.BlockSpec((1, HBLK, CROP), lambda b, h: (b, h, 0)),
            pl.BlockSpec((1, HBLK, CROP), lambda b, h: (b, h, 0)),
            pl.BlockSpec((4, HBLK, CROP), lambda b, h: (0, b * NHB + h, 0)),
            pl.BlockSpec((4, HBLK, CROP), lambda b, h: (0, b * NHB + h, 0)),
        ],
        out_shape=[
            jax.ShapeDtypeStruct((B, CROP, CROP), jnp.float32),
            jax.ShapeDtypeStruct((B, CROP, CROP), jnp.float32),
            jax.ShapeDtypeStruct((4, NROW, CROP), jnp.int32),
            jax.ShapeDtypeStruct((4, NROW, CROP), jnp.float32),
        ],
        compiler_params=pltpu.CompilerParams(
            dimension_semantics=("parallel", "parallel")),
        interpret=interpret,
    )(scal, rot, base)


def _inv_body(scal_ref, rot_ref, ug_ref, vg_ref):
    b = pl.program_id(0)
    hb = pl.program_id(1)
    inv = scal_ref[4]
    ii = lax.broadcasted_iota(jnp.int32, (HBLK, WE), 0).astype(jnp.float32) + (hb * HBLK).astype(jnp.float32)
    jj = lax.broadcasted_iota(jnp.int32, (HBLK, WE), 1).astype(jnp.float32)
    ue = (jj + 0.5) / WE * 2.0 - 1.0
    ve = (ii + 0.5) / HE * 2.0 - 1.0
    th = -ue * math.pi
    ph = (ve * math.pi) / 2.0
    sph = jnp.sin(ph)
    cph = jnp.cos(ph)
    sth = jnp.sin(th)
    cth = jnp.cos(th)
    ex = _bf(sth * cph)
    ey = _bf(sph)
    ez = _bf(cth * cph)
    r00 = rot_ref[b, 0]; r01 = rot_ref[b, 1]; r02 = rot_ref[b, 2]
    r10 = rot_ref[b, 3]; r11 = rot_ref[b, 4]; r12 = rot_ref[b, 5]
    r20 = rot_ref[b, 6]; r21 = rot_ref[b, 7]; r22 = rot_ref[b, 8]
    x = _kahan3(r00 * ex, r10 * ey, r20 * ez)
    y = _kahan3(r01 * ex, r11 * ey, r21 * ez)
    z = _kahan3(r02 * ex, r12 * ey, r22 * ez)
    eps = 1e-6
    valid = z > eps
    zc = jnp.where(valid, z, eps)
    ug_ref[0] = jnp.where(valid, x / zc, inv)
    vg_ref[0] = jnp.where(valid, y / zc, inv)


def _inv_call(scal, rot, interpret=False):
    return pl.pallas_call(
        _inv_body,
        grid=(B, HE // HBLK),
        in_specs=[
            pl.BlockSpec(memory_space=pltpu.SMEM),
            pl.BlockSpec(memory_space=pltpu.SMEM),
        ],
        out_specs=[
            pl.BlockSpec((1, HBLK, WE), lambda b, h: (b, h, 0)),
            pl.BlockSpec((1, HBLK, WE), lambda b, h: (b, h, 0)),
        ],
        out_shape=[
            jax.ShapeDtypeStruct((B, HE, WE), jnp.float32),
            jax.ShapeDtypeStruct((B, HE, WE), jnp.float32),
        ],
        compiler_params=pltpu.CompilerParams(
            dimension_semantics=("parallel", "parallel")),
        interpret=interpret,
    )(scal, rot)


def _sc_body(table, idxh, wh, img_o, ima_o, dis_o, dma_o,
             iw, wv, gbuf, obuf, gsem, psem, osem):
    cid = lax.axis_index("c")
    sid = lax.axis_index("s")
    wid = sid * NC + cid
    b = wid // 2
    row0 = wid * ROWS_PER_TILE
    last = ROWS_PER_TILE - 1
    lane = lax.broadcasted_iota(jnp.int32, (16,), 0)

    def prefetch(k, slot):
        rowg = row0 + k
        a = pltpu.async_copy(idxh.at[:, rowg, :], iw.at[slot], psem)
        bcp = pltpu.async_copy(wh.at[:, rowg, :], wv.at[slot], psem)
        return a, bcp

    def issue_gathers(slot):
        cps = []
        for t in range(4):
            for j in range(4):
                cps.append(pltpu.async_copy(
                    table.at[iw.at[slot, t, pl.ds(j * 128, 128)]],
                    gbuf.at[slot, t].at[pl.ds(j * 128, 128)],
                    gsem))
        return cps

    pltpu.sync_copy(idxh.at[:, row0, :], iw.at[0])
    pltpu.sync_copy(wh.at[:, row0, :], wv.at[0])
    issue_gathers(0)
    prefetch(jnp.minimum(1, last), 1)

    def process(k, cur, nxt):
        q = (row0 + k - b * CROP) * CROP
        for _ in range(16):
            pltpu.make_async_copy(table.at[pl.ds(0, 128)],
                                  gbuf.at[0, 0].at[pl.ds(0, 128)], gsem).wait()
        pltpu.make_async_copy(idxh.at[:, 0, :], iw.at[0], psem).wait()
        pltpu.make_async_copy(wh.at[:, 0, :], wv.at[0], psem).wait()
        issue_gathers(nxt)
        for p16 in range(CROP // 16):
            w0 = wv[cur, 0, pl.ds(p16 * 16, 16)]
            w1 = wv[cur, 1, pl.ds(p16 * 16, 16)]
            w2 = wv[cur, 2, pl.ds(p16 * 16, 16)]
            w3 = wv[cur, 3, pl.ds(p16 * 16, 16)]
            pidx = lane + (p16 * 16)
            gcur = gbuf.at[cur]
            for c in range(6):
                cc = jnp.full((16,), c, jnp.int32)
                g0 = plsc.load_gather(gcur, [jnp.full((16,), 0, jnp.int32), pidx, cc])
                g1 = plsc.load_gather(gcur, [jnp.full((16,), 1, jnp.int32), pidx, cc])
                g2 = plsc.load_gather(gcur, [jnp.full((16,), 2, jnp.int32), pidx, cc])
                g3 = plsc.load_gather(gcur, [jnp.full((16,), 3, jnp.int32), pidx, cc])
                acc = w0 * g0 + w1 * g1 + w2 * g2 + w3 * g3
                obuf[cur, c, pl.ds(p16 * 16, 16)] = acc
        prefetch(jnp.minimum(k + 2, last), cur)
        @pl.when(k > 0)
        def _():
            for c in range(6):
                pltpu.make_async_copy(wh.at[0, 0, :], obuf.at[0, 0], osem).wait()
        pltpu.async_copy(obuf.at[cur, 0], img_o.at[pl.ds((b * 3 + 0) * HW + q, CROP)], osem)
        pltpu.async_copy(obuf.at[cur, 1], img_o.at[pl.ds((b * 3 + 1) * HW + q, CROP)], osem)
        pltpu.async_copy(obuf.at[cur, 2], img_o.at[pl.ds((b * 3 + 2) * HW + q, CROP)], osem)
        pltpu.async_copy(obuf.at[cur, 3], ima_o.at[pl.ds(b * HW + q, CROP)], osem)
        pltpu.async_copy(obuf.at[cur, 4], dis_o.at[pl.ds(b * HW + q, CROP)], osem)
        pltpu.async_copy(obuf.at[cur, 5], dma_o.at[pl.ds(b * HW + q, CROP)], osem)

    def body(m, carry):
        process(2 * m, 0, 1)
        process(2 * m + 1, 1, 0)
        return carry

    lax.fori_loop(0, ROWS_PER_TILE // 2, body, 0)
    pltpu.make_async_copy(idxh.at[:, 0, :], iw.at[0], psem).wait()
    pltpu.make_async_copy(wh.at[:, 0, :], wv.at[0], psem).wait()
    for _ in range(16):
        pltpu.make_async_copy(table.at[pl.ds(0, 128)],
                              gbuf.at[0, 0].at[pl.ds(0, 128)], gsem).wait()
    for _ in range(6):
        pltpu.make_async_copy(wh.at[0, 0, :], obuf.at[0, 0], osem).wait()


def _sc_call(table, idxh, wh, interpret=False):
    mesh = plsc.VectorSubcoreMesh(core_axis_name="c", subcore_axis_name="s",
                                  num_cores=NC, num_subcores=NS)
    fn = pl.kernel(
        _sc_body,
        out_type=[
            jax.ShapeDtypeStruct((B * 3 * HW,), jnp.float32),
            jax.ShapeDtypeStruct((NPIX,), jnp.float32),
            jax.ShapeDtypeStruct((NPIX,), jnp.float32),
            jax.ShapeDtypeStruct((NPIX,), jnp.float32),
        ],
        mesh=mesh,
        scratch_types=[
            pltpu.VMEM((2, 4, CROP), jnp.int32),
            pltpu.VMEM((2, 4, CROP), jnp.float32),
            pltpu.VMEM((2, 4, CROP, ROWD), jnp.float32),
            pltpu.VMEM((2, 6, CROP), jnp.float32),
            pltpu.SemaphoreType.DMA,
            pltpu.SemaphoreType.DMA,
            pltpu.SemaphoreType.DMA,
        ],
        compiler_params=pltpu.CompilerParams(needs_layout_passes=False,
                                             use_tc_tiling_on_sc=False),
        interpret=interpret,
    )
    return fn(table, idxh, wh)


def kernel(image_indices, yaws, pitches, fov_x, fov_y, width, height,
           image, image_mask, distance, distance_mask, invalid_number=2.0):
    f32 = jnp.float32
    yr = (yaws.astype(f32) * math.pi) / 180.0
    pr = ((-1.0 * pitches.astype(f32)) * math.pi) / 180.0
    cy, sy = jnp.cos(yr), jnp.sin(yr)
    cp, sp = jnp.cos(pr), jnp.sin(pr)
    bcy, bsy, bcp, bsp = _bf_hard(cy), _bf_hard(sy), _bf_hard(cp), _bf_hard(sp)
    zero = jnp.zeros_like(cp)
    rot = jnp.stack([bcy * bcp, -bsy, bcy * bsp,
                     bsy * bcp, bcy, bsy * bsp,
                     -bsp, zero, bcp], axis=-1)
    rot_bf = _bf_hard(rot)
    fovx_r = (jnp.asarray(fov_x, f32) * math.pi) / 180.0
    fovy_r = (jnp.asarray(fov_y, f32) * math.pi) / 180.0
    scal = jnp.stack([
        jnp.tan(fovx_r / 2.0),
        jnp.tan(fovy_r / 2.0),
        jnp.asarray(width, f32),
        jnp.asarray(height, f32),
        jnp.asarray(invalid_number, f32),
        jnp.zeros((), f32), jnp.zeros((), f32), jnp.zeros((), f32),
    ])
    base = image_indices.astype(jnp.int32) * (HE * WE)

    table = _table_call(image.astype(f32), image_mask.astype(f32),
                        distance.astype(f32), distance_mask.astype(f32))

    u, v, idxh, wh = _taps_call(scal, rot_bf, base)
    img_f, ima_f, dis_f, dma_f = _sc_call(table, idxh, wh)
    ug, vg = _inv_call(scal, rot_bf)

    image_out = img_f.reshape(B, 3, CROP, CROP)
    image_mask_out = ima_f.reshape(B, 1, CROP, CROP)
    distance_out = dis_f.reshape(B, 1, CROP, CROP)
    distance_mask_out = dma_f.reshape(B, 1, CROP, CROP)
    grid = jnp.stack([u, v], axis=-1)
    inverse_grid = jnp.stack([ug, vg], axis=-1)
    return (image_out, image_mask_out, distance_out, distance_mask_out,
            grid, inverse_grid)
